# Initial kernel scaffold; baseline (speedup 1.0000x reference)
import jax
import jax.numpy as jnp
from jax.experimental import pallas as pl


def kernel(x, target):
    raise NotImplementedError("write your pallas kernel here")


# speedup vs baseline: 1.6563x; 1.6563x over previous
import functools
import math

import jax
import jax.numpy as jnp
from jax.experimental import pallas as pl
from jax.experimental.pallas import tpu as pltpu

_V = 100000
_B = 1024
_S = 0.1 / (_V - 2)
_C = 0.9
_ENT = _C * math.log(_C) + (_V - 2) * _S * math.log(_S)

_BV = 2048
_NK = (_V + _BV - 1) // _BV


def _body(t_ref, x_ref, o_ref):
    k = pl.program_id(0)
    t = t_ref[...]
    mask = t != 0
    roww = jnp.where(mask, -_S, 0.0).astype(jnp.float32)
    x = x_ref[...]
    cols = k * _BV + jax.lax.broadcasted_iota(jnp.int32, (_B, _BV), 1)
    w = jnp.where(cols == t, jnp.float32(-_C), roww)

    @pl.when(k == 0)
    def _first():
        w0 = jnp.where(cols == 0, 0.0, w)
        count = jnp.sum(mask.astype(jnp.float32))
        o_ref[0, 0] = jnp.float32(_ENT) * count + jnp.sum(w0 * x)

    @pl.when((k > 0) & (k < _NK - 1))
    def _mid():
        o_ref[0, 0] += jnp.sum(w * x)

    @pl.when(k == _NK - 1)
    def _last():
        o_ref[0, 0] += jnp.sum(jnp.where(cols < _V, w * x, 0.0))


@jax.jit
def kernel(x, target):
    t2 = target.astype(jnp.int32).reshape(_B, 1)
    out = pl.pallas_call(
        _body,
        grid=(_NK,),
        in_specs=[
            pl.BlockSpec((_B, 1), lambda k: (0, 0)),
            pl.BlockSpec((_B, _BV), lambda k: (0, k)),
        ],
        out_specs=pl.BlockSpec(memory_space=pltpu.SMEM),
        out_shape=jax.ShapeDtypeStruct((1, 1), jnp.float32),
        compiler_params=pltpu.CompilerParams(
            dimension_semantics=("arbitrary",),
        ),
    )(t2, x)
    return out[0, 0]
